# Initial kernel scaffold; baseline (speedup 1.0000x reference)
import jax
import jax.numpy as jnp
from jax.experimental import pallas as pl


def kernel(x, bank_mag, bank_freq, bank_phase, bank_down, router_W, router_bias, norm_weight):
    raise NotImplementedError("write your pallas kernel here")


# speedup vs baseline: 1.3562x; 1.3562x over previous
import functools

import jax
import jax.numpy as jnp
from jax.experimental import pallas as pl
from jax.experimental.pallas import tpu as pltpu

B, T, D, H, N = 1, 2048, 768, 64, 64


def _moe_body(x_ref, mag_ref, freq_ref, phase_ref, down_ref, rw_ref, rb_ref,
              nw_ref, out_ref, acc_ref, top_ref):
    e = pl.program_id(0)

    @pl.when(e == 0)
    def _():
        scores = jnp.dot(x_ref[:], rw_ref[:].T,
                         preferred_element_type=jnp.float32) + rb_ref[:]
        top_ref[:] = jnp.argmax(scores, axis=-1, keepdims=True).astype(jnp.int32)

    mag = jnp.dot(x_ref[:], mag_ref[0], preferred_element_type=jnp.float32)
    freq = jnp.dot(x_ref[:], freq_ref[0], preferred_element_type=jnp.float32)
    hidden = jnp.tanh(mag) * jnp.cos(
        jax.nn.softplus(freq) + 0.1 + phase_ref[0, 0])
    o = jnp.dot(hidden, down_ref[0], preferred_element_type=jnp.float32)
    contrib = jnp.where(top_ref[:] == e, o, 0.0)

    @pl.when(e == 0)
    def _():
        acc_ref[:] = contrib

    @pl.when(e > 0)
    def _():
        acc_ref[:] += contrib

    @pl.when(e == N - 1)
    def _():
        a = acc_ref[:]
        var = jnp.mean(a * a, axis=-1, keepdims=True)
        out_ref[:] = a * jax.lax.rsqrt(var + 1e-6) * nw_ref[:]


@functools.partial(jax.jit, static_argnames=())
def kernel(x, bank_mag, bank_freq, bank_phase, bank_down, router_W,
           router_bias, norm_weight):
    xf = x.reshape(T, D)
    phase3 = bank_phase.reshape(N, 1, H)
    rb = router_bias.reshape(1, N)
    nw = norm_weight.reshape(1, D)

    out = pl.pallas_call(
        _moe_body,
        grid=(N,),
        in_specs=[
            pl.BlockSpec((T, D), lambda e: (0, 0)),
            pl.BlockSpec((1, D, H), lambda e: (e, 0, 0)),
            pl.BlockSpec((1, D, H), lambda e: (e, 0, 0)),
            pl.BlockSpec((1, 1, H), lambda e: (e, 0, 0)),
            pl.BlockSpec((1, H, D), lambda e: (e, 0, 0)),
            pl.BlockSpec((N, D), lambda e: (0, 0)),
            pl.BlockSpec((1, N), lambda e: (0, 0)),
            pl.BlockSpec((1, D), lambda e: (0, 0)),
        ],
        out_specs=pl.BlockSpec((T, D), lambda e: (0, 0)),
        out_shape=jax.ShapeDtypeStruct((T, D), jnp.float32),
        scratch_shapes=[
            pltpu.VMEM((T, D), jnp.float32),
            pltpu.VMEM((T, 1), jnp.int32),
        ],
    )(xf, bank_mag, bank_freq, phase3, bank_down, router_W, rb, nw)
    return out.reshape(B, T, D)
